# baseline (speedup 1.0000x reference)
import functools

import jax
import jax.numpy as jnp
from jax import lax
from jax.experimental import pallas as pl
from jax.experimental.pallas import tpu as pltpu
from jax.experimental.pallas import tpu_sc as plsc

_NUM_WORKERS = 32
_NBUF = 3


def kernel(inputs, token_table, pos_table):
    B, L = inputs.shape
    V, E = token_table.shape
    BBLK = B // _NUM_WORKERS

    inputs_t = jnp.swapaxes(inputs, 0, 1)

    mesh = plsc.VectorSubcoreMesh(core_axis_name="c", subcore_axis_name="s")

    @functools.partial(
        pl.kernel,
        mesh=mesh,
        compiler_params=pltpu.CompilerParams(use_tc_tiling_on_sc=False,
                                             needs_layout_passes=False),
        out_type=jax.ShapeDtypeStruct((L, E, B), jnp.float32),
        scratch_types=[
            pltpu.VMEM((L, BBLK), jnp.int32),
            pltpu.VMEM((L, E), jnp.float32),
            [pltpu.VMEM((BBLK, E), jnp.float32)] * _NBUF,
            [pltpu.VMEM((E, BBLK), jnp.float32)] * _NBUF,
            [pltpu.SemaphoreType.DMA] * _NBUF,
            [pltpu.SemaphoreType.DMA] * _NBUF,
        ],
    )
    def emb_kernel(inputs_hbm, table_hbm, pos_hbm, out_hbm,
                   idx_v, pos_v, gbufs, tbufs, gsems, wsems):
        wid = lax.axis_index("s") * 2 + lax.axis_index("c")
        bbase = wid * BBLK

        pltpu.sync_copy(inputs_hbm.at[:, pl.ds(bbase, BBLK)], idx_v)
        pltpu.sync_copy(pos_hbm, pos_v)

        def gather_cp(l, k):
            return pltpu.make_async_copy(
                table_hbm.at[idx_v.at[l, :]], gbufs[k], gsems[k])

        def wb_cp(l, k):
            return pltpu.make_async_copy(
                tbufs[k], out_hbm.at[l, :, pl.ds(bbase, BBLK)], wsems[k])

        for k in range(_NBUF - 1):
            gather_cp(k, k).start()

        lanes = jnp.arange(16, dtype=jnp.int32)

        def body(l, carry):
            k = lax.rem(l, _NBUF)

            @pl.when(l + _NBUF - 1 < L)
            def _fire_ahead():
                kn = lax.rem(l + _NBUF - 1, _NBUF)
                for kk in range(_NBUF):
                    @pl.when(kn == kk)
                    def _fire():
                        gather_cp(l + _NBUF - 1, kk).start()

            for kk in range(_NBUF):
                @pl.when(k == kk)
                def _work():
                    gather_cp(l, kk).wait()

                    @pl.when(l >= _NBUF)
                    def _drain_wb():
                        wb_cp(l, kk).wait()

                    def erow(e, carry2):
                        efull = jnp.full((16,), e, dtype=jnp.int32)
                        lfull = jnp.full((16,), l, dtype=jnp.int32)
                        splat = plsc.load_gather(pos_v, [lfull, efull])
                        for g in range(8):
                            col = plsc.load_gather(
                                gbufs[kk], [lanes + (16 * g), efull])
                            tbufs[kk][e, pl.ds(16 * g, 16)] = col + splat
                        return carry2

                    lax.fori_loop(0, E, erow, 0)
                    wb_cp(l, kk).start()

            return carry

        lax.fori_loop(0, L, body, 0)
        for k in range(_NBUF):
            wb_cp(L - _NBUF + k, k).wait()

    out = emb_kernel(inputs_t, token_table, pos_table)
    return jnp.transpose(out, (2, 0, 1))
